# Initial kernel scaffold; baseline (speedup 1.0000x reference)
import jax
import jax.numpy as jnp
from jax.experimental import pallas as pl


def kernel(input_value, mask_tensor, weight, bias):
    raise NotImplementedError("write your pallas kernel here")


# speedup vs baseline: 17.5301x; 17.5301x over previous
import jax
import jax.numpy as jnp
from jax import lax
from jax.experimental import pallas as pl
from jax.experimental.pallas import tpu as pltpu
from jax.experimental.pallas import tpu_sc as plsc

B, S, M, H, V = 1024, 50, 20, 64, 100000
N = B * S
NC, NS = 2, 16
NW = NC * NS
TOK_W = N // NW
T = 16
CHUNKS = TOK_W // T
RPC = T * M
IDX_MINOR = 64
IDX_ROWS = RPC // IDX_MINOR


def _sc_kernel(x_hbm, idx_hbm, w_hbm, b_hbm, out_hbm,
               bias_v, idx_v, rows_v, x_v, out_v, acc_t, sem):
    c_id = lax.axis_index("c")
    s_id = lax.axis_index("s")
    wid = s_id * NC + c_id
    pltpu.sync_copy(b_hbm, bias_v)
    tok_base = wid * TOK_W
    lane = lax.iota(jnp.int32, 16)

    def chunk_body(c, carry):
        tok0 = tok_base + c * T
        pltpu.sync_copy(idx_hbm.at[pl.ds(tok0 * M, RPC)], idx_v)
        pltpu.sync_copy(x_hbm.at[pl.ds(tok0, T)], x_v)
        copies = [
            pltpu.async_copy(w_hbm.at[idx_v.at[pl.ds(j * IDX_MINOR, IDX_MINOR)]],
                             rows_v.at[pl.ds(j * IDX_MINOR, IDX_MINOR)], sem)
            for j in range(IDX_ROWS)
        ]
        for cp in copies:
            cp.wait()

        def tok_body(t, tc):
            xs = [x_v[t, pl.ds(k * 16, 16)] for k in range(4)]
            for m in range(M):
                r = t * M + m
                acc = xs[0] * rows_v[r, pl.ds(0, 16)]
                for k in range(1, 4):
                    acc = acc + xs[k] * rows_v[r, pl.ds(k * 16, 16)]
                sidx = lane * RPC + r
                plsc.store_scatter(acc_t, [sidx], acc)
            return tc

        lax.fori_loop(0, T, tok_body, 0)

        def grp_body(g, gc):
            r0 = g * 16
            colsum = acc_t[pl.ds(r0, 16)]
            for l in range(1, 16):
                colsum = colsum + acc_t[pl.ds(l * RPC + r0, 16)]
            idxvec = idx_v[pl.ds(r0, 16)]
            biasvals = plsc.load_gather(bias_v, [idxvec])
            out_v[pl.ds(r0, 16)] = jnp.maximum(colsum + biasvals, 0.0)
            return gc

        lax.fori_loop(0, RPC // 16, grp_body, 0)
        pltpu.sync_copy(out_v, out_hbm.at[pl.ds(tok0 * M, RPC)])
        return carry

    lax.fori_loop(0, CHUNKS, chunk_body, 0)


@jax.jit
def _run(x2d, idx2d, weight, bias):
    f = pl.kernel(
        _sc_kernel,
        out_type=jax.ShapeDtypeStruct((N * M,), jnp.float32),
        mesh=plsc.VectorSubcoreMesh(core_axis_name="c", subcore_axis_name="s"),
        compiler_params=pltpu.CompilerParams(
            needs_layout_passes=False, use_tc_tiling_on_sc=False),
        scratch_types=[
            pltpu.VMEM((V,), jnp.float32),
            pltpu.VMEM((RPC,), jnp.int32),
            pltpu.VMEM((RPC, H), jnp.float32),
            pltpu.VMEM((T, H), jnp.float32),
            pltpu.VMEM((RPC,), jnp.float32),
            pltpu.VMEM((16 * RPC,), jnp.float32),
            pltpu.SemaphoreType.DMA,
        ],
    )
    return f(x2d, idx2d, weight, bias)


def kernel(input_value, mask_tensor, weight, bias):
    x2d = input_value.reshape(N, H)
    idx1d = jnp.asarray(mask_tensor, jnp.int32).reshape(N * M)
    out = _run(x2d, idx1d, weight, bias)
    return out.reshape(B, S, M)


# speedup vs baseline: 21.6316x; 1.2340x over previous
import jax
import jax.numpy as jnp
from jax import lax
from jax.experimental import pallas as pl
from jax.experimental.pallas import tpu as pltpu
from jax.experimental.pallas import tpu_sc as plsc

B, S, M, H, V = 1024, 50, 20, 64, 100000
N = B * S
NC, NS = 2, 16
NW = NC * NS
TOK_W = N // NW
T = 16
CHUNKS = TOK_W // T
RPC = T * M
SPLITS = ((0, 128), (128, 128), (256, 64))
NGRP = RPC // 16


def _sc_kernel(x_hbm, idx_hbm, w_hbm, b_hbm, out_hbm,
               bias_v, idx0, idx1, rows0, rows1, x0, x1, out0, out1, acc_t,
               sg0, sg1, si0, si1, sx0, sx1, so0, so1):
    c_id = lax.axis_index("c")
    s_id = lax.axis_index("s")
    wid = s_id * NC + c_id
    tok_base = wid * TOK_W
    lane = lax.iota(jnp.int32, 16)

    idx_v = (idx0, idx1)
    rows_v = (rows0, rows1)
    x_v = (x0, x1)
    out_v = (out0, out1)
    sg = (sg0, sg1)
    si = (si0, si1)
    sx = (sx0, sx1)
    so = (so0, so1)

    def idx_copy(bi, tok):
        return pltpu.make_async_copy(
            idx_hbm.at[pl.ds(tok * M, RPC)], idx_v[bi], si[bi])

    def x_copy(bi, tok):
        return pltpu.make_async_copy(
            x_hbm.at[pl.ds(tok, T)], x_v[bi], sx[bi])

    def out_copy(bi, tok):
        return pltpu.make_async_copy(
            out_v[bi], out_hbm.at[pl.ds(tok * M, RPC)], so[bi])

    def g_copies(bi):
        return [
            pltpu.make_async_copy(
                w_hbm.at[idx_v[bi].at[pl.ds(o, l)]],
                rows_v[bi].at[pl.ds(o, l)], sg[bi])
            for (o, l) in SPLITS
        ]

    pltpu.sync_copy(b_hbm, bias_v)

    pltpu.sync_copy(idx_hbm.at[pl.ds(tok_base * M, RPC)], idx_v[0])
    for cp in g_copies(0):
        cp.start()
    idx_copy(1, tok_base + T).start()
    x_copy(0, tok_base).start()
    x_copy(1, tok_base + T).start()

    def compute_chunk(bi):
        rows = rows_v[bi]
        xv = x_v[bi]

        def tok_body(t, tc):
            xa = xv[t, pl.ds(0, 32)]
            xb = xv[t, pl.ds(32, 32)]
            for m in range(M):
                r = t * M + m
                p = xa * rows[r, pl.ds(0, 32)] + xb * rows[r, pl.ds(32, 32)]
                lo, hi = plsc.unpack(p, format=plsc.PackFormat.INTERLEAVED)
                plsc.store_scatter(acc_t, [lane * RPC + r], lo + hi)
            return tc

        lax.fori_loop(0, T, tok_body, 0)

        def grp_body(g, gc):
            r0 = g * 16
            colsum = acc_t[pl.ds(r0, 16)]
            for l in range(1, 16):
                colsum = colsum + acc_t[pl.ds(l * RPC + r0, 16)]
            biasvals = plsc.load_gather(bias_v, [idx_v[bi][pl.ds(r0, 16)]])
            out_v[bi][pl.ds(r0, 16)] = jnp.maximum(colsum + biasvals, 0.0)
            return gc

        lax.fori_loop(0, NGRP, grp_body, 0)

    def pair_body(pp, carry):
        for b in (0, 1):
            nb = 1 - b
            c = 2 * pp + b
            tok_c = tok_base + c * T

            def fire_next():
                idx_copy(nb, tok_c + T).wait()
                for cp in g_copies(nb):
                    cp.start()
            if b == 0:
                fire_next()
            else:
                pl.when(pp < CHUNKS // 2 - 1)(fire_next)

            for cp in g_copies(b):
                cp.wait()

            pl.when(pp > 0)(lambda: out_copy(b, tok_c - 2 * T).wait())

            x_copy(b, tok_c).wait()

            compute_chunk(b)
            out_copy(b, tok_c).start()

            def prefetch():
                idx_copy(b, tok_c + 2 * T).start()
                x_copy(b, tok_c + 2 * T).start()
            pl.when(pp < CHUNKS // 2 - 1)(prefetch)
        return carry

    lax.fori_loop(0, CHUNKS // 2, pair_body, 0)

    out_copy(0, tok_base + (CHUNKS - 2) * T).wait()
    out_copy(1, tok_base + (CHUNKS - 1) * T).wait()


@jax.jit
def _run(x2d, idx1d, weight, bias):
    f = pl.kernel(
        _sc_kernel,
        out_type=jax.ShapeDtypeStruct((N * M,), jnp.float32),
        mesh=plsc.VectorSubcoreMesh(core_axis_name="c", subcore_axis_name="s"),
        compiler_params=pltpu.CompilerParams(
            needs_layout_passes=False, use_tc_tiling_on_sc=False),
        scratch_types=[
            pltpu.VMEM((V,), jnp.float32),
            pltpu.VMEM((RPC,), jnp.int32),
            pltpu.VMEM((RPC,), jnp.int32),
            pltpu.VMEM((RPC, H), jnp.bfloat16),
            pltpu.VMEM((RPC, H), jnp.bfloat16),
            pltpu.VMEM((T, H), jnp.bfloat16),
            pltpu.VMEM((T, H), jnp.bfloat16),
            pltpu.VMEM((RPC,), jnp.float32),
            pltpu.VMEM((RPC,), jnp.float32),
            pltpu.VMEM((16 * RPC,), jnp.float32),
        ] + [pltpu.SemaphoreType.DMA] * 8,
    )
    return f(x2d, idx1d, weight, bias)


def kernel(input_value, mask_tensor, weight, bias):
    x2d = input_value.reshape(N, H).astype(jnp.bfloat16)
    idx1d = jnp.asarray(mask_tensor, jnp.int32).reshape(N * M)
    w_bf = weight.astype(jnp.bfloat16)
    out = _run(x2d, idx1d, w_bf, bias)
    return out.reshape(B, S, M)


# speedup vs baseline: 27.0753x; 1.2517x over previous
import jax
import jax.numpy as jnp
from jax import lax
from jax.experimental import pallas as pl
from jax.experimental.pallas import tpu as pltpu
from jax.experimental.pallas import tpu_sc as plsc

B, S, M, H, V = 1024, 50, 20, 64, 100000
N = B * S
NC, NS = 2, 16
NW = NC * NS
TOK_W = N // NW
T = 16
CHUNKS = TOK_W // T
RPC = T * M
SPLITS = ((0, 128), (128, 128), (256, 64))
NGRP = RPC // 16


def _sc_kernel(x_hbm, idx_hbm, w_hbm, b_hbm, out_hbm,
               bias_v, idx0, idx1, rows0, rows1, x0, x1, out0, out1, acc_t,
               sg0, sg1, si0, si1, sx0, sx1, so0, so1):
    c_id = lax.axis_index("c")
    s_id = lax.axis_index("s")
    wid = s_id * NC + c_id
    tok_base = wid * TOK_W
    lane = lax.iota(jnp.int32, 16)

    idx_v = (idx0, idx1)
    rows_v = (rows0, rows1)
    x_v = (x0, x1)
    out_v = (out0, out1)
    sg = (sg0, sg1)
    si = (si0, si1)
    sx = (sx0, sx1)
    so = (so0, so1)

    def idx_copy(bi, tok):
        return pltpu.make_async_copy(
            idx_hbm.at[pl.ds(tok * M, RPC)], idx_v[bi], si[bi])

    def x_copy(bi, tok):
        return pltpu.make_async_copy(
            x_hbm.at[pl.ds(tok, T)], x_v[bi], sx[bi])

    def out_copy(bi, tok):
        return pltpu.make_async_copy(
            out_v[bi], out_hbm.at[pl.ds(tok * M, RPC)], so[bi])

    def g_copies(bi):
        return [
            pltpu.make_async_copy(
                w_hbm.at[idx_v[bi].at[pl.ds(o, l)]],
                rows_v[bi].at[pl.ds(o, l)], sg[bi])
            for (o, l) in SPLITS
        ]

    pltpu.sync_copy(b_hbm, bias_v)

    pltpu.sync_copy(idx_hbm.at[pl.ds(tok_base * M, RPC)], idx_v[0])
    for cp in g_copies(0):
        cp.start()
    idx_copy(1, tok_base + T).start()
    x_copy(0, tok_base).start()
    x_copy(1, tok_base + T).start()

    lane16 = lane * 16

    def compute_chunk(bi):
        rows = rows_v[bi]
        xv = x_v[bi]

        def tok_body(tt, tc):
            t0 = tt * 2
            xs = [(xv[t0 + i, pl.ds(0, 32)], xv[t0 + i, pl.ds(32, 32)])
                  for i in range(2)]
            for m in range(M):
                for i in range(2):
                    r = (t0 + i) * M + m
                    xa, xb = xs[i]
                    p = (xa * rows[r, pl.ds(0, 32)]
                         + xb * rows[r, pl.ds(32, 32)])
                    lo, hi = plsc.unpack(
                        p, format=plsc.PackFormat.INTERLEAVED)
                    acc_t[pl.ds(r * 16, 16)] = lo + hi
            return tc

        lax.fori_loop(0, T // 2, tok_body, 0)

        def grp_body(g, gc):
            vbase = lane16 + g * 256
            vals = [plsc.load_gather(acc_t, [vbase + l]) for l in range(16)]
            while len(vals) > 1:
                vals = [vals[i] + vals[i + 1] for i in range(0, len(vals), 2)]
            r0 = g * 16
            biasvals = plsc.load_gather(bias_v, [idx_v[bi][pl.ds(r0, 16)]])
            out_v[bi][pl.ds(r0, 16)] = jnp.maximum(vals[0] + biasvals, 0.0)
            return gc

        lax.fori_loop(0, NGRP, grp_body, 0)

    def pair_body(pp, carry):
        for b in (0, 1):
            nb = 1 - b
            c = 2 * pp + b
            tok_c = tok_base + c * T

            def fire_next():
                idx_copy(nb, tok_c + T).wait()
                for cp in g_copies(nb):
                    cp.start()
            if b == 0:
                fire_next()
            else:
                pl.when(pp < CHUNKS // 2 - 1)(fire_next)

            for cp in g_copies(b):
                cp.wait()

            pl.when(pp > 0)(lambda: out_copy(b, tok_c - 2 * T).wait())

            x_copy(b, tok_c).wait()

            compute_chunk(b)
            out_copy(b, tok_c).start()

            def prefetch():
                idx_copy(b, tok_c + 2 * T).start()
                x_copy(b, tok_c + 2 * T).start()
            pl.when(pp < CHUNKS // 2 - 1)(prefetch)
        return carry

    lax.fori_loop(0, CHUNKS // 2, pair_body, 0)

    out_copy(0, tok_base + (CHUNKS - 2) * T).wait()
    out_copy(1, tok_base + (CHUNKS - 1) * T).wait()


@jax.jit
def _run(x2d, idx1d, weight, bias):
    f = pl.kernel(
        _sc_kernel,
        out_type=jax.ShapeDtypeStruct((N * M,), jnp.float32),
        mesh=plsc.VectorSubcoreMesh(core_axis_name="c", subcore_axis_name="s"),
        compiler_params=pltpu.CompilerParams(
            needs_layout_passes=False, use_tc_tiling_on_sc=False),
        scratch_types=[
            pltpu.VMEM((V,), jnp.float32),
            pltpu.VMEM((RPC,), jnp.int32),
            pltpu.VMEM((RPC,), jnp.int32),
            pltpu.VMEM((RPC, H), jnp.bfloat16),
            pltpu.VMEM((RPC, H), jnp.bfloat16),
            pltpu.VMEM((T, H), jnp.bfloat16),
            pltpu.VMEM((T, H), jnp.bfloat16),
            pltpu.VMEM((RPC,), jnp.float32),
            pltpu.VMEM((RPC,), jnp.float32),
            pltpu.VMEM((16 * RPC,), jnp.float32),
        ] + [pltpu.SemaphoreType.DMA] * 8,
    )
    return f(x2d, idx1d, weight, bias)


def kernel(input_value, mask_tensor, weight, bias):
    x2d = input_value.reshape(N, H).astype(jnp.bfloat16)
    idx1d = jnp.asarray(mask_tensor, jnp.int32).reshape(N * M)
    w_bf = weight.astype(jnp.bfloat16)
    out = _run(x2d, idx1d, w_bf, bias)
    return out.reshape(B, S, M)


# speedup vs baseline: 37.5892x; 1.3883x over previous
import jax
import jax.numpy as jnp
from jax import lax
from jax.experimental import pallas as pl
from jax.experimental.pallas import tpu as pltpu
from jax.experimental.pallas import tpu_sc as plsc

B, S, M, H, V = 1024, 50, 20, 64, 100000
N = B * S
NC, NS = 2, 16
NW = NC * NS
TOK_W = N // NW
T = 16
CHUNKS = TOK_W // T
RPC = T * M
SPLITS = ((0, 128), (128, 128), (256, 64))
NGRP = RPC // 16


def _sc_kernel(x_hbm, idx_hbm, w_hbm, b_hbm, out_hbm,
               bias_v, idx0, idx1, rows0, rows1, x0, x1, out0, out1, acc_t,
               sg0, sg1, si0, si1, sx0, sx1, so0, so1):
    c_id = lax.axis_index("c")
    s_id = lax.axis_index("s")
    wid = s_id * NC + c_id
    tok_base = wid * TOK_W
    lane = lax.iota(jnp.int32, 16)

    idx_v = (idx0, idx1)
    rows_v = (rows0, rows1)
    x_v = (x0, x1)
    out_v = (out0, out1)
    sg = (sg0, sg1)
    si = (si0, si1)
    sx = (sx0, sx1)
    so = (so0, so1)

    def idx_copy(bi, tok):
        return pltpu.make_async_copy(
            idx_hbm.at[pl.ds(tok * M, RPC)], idx_v[bi], si[bi])

    def x_copy(bi, tok):
        return pltpu.make_async_copy(
            x_hbm.at[pl.ds(tok, T)], x_v[bi], sx[bi])

    def out_copy(bi, tok):
        return pltpu.make_async_copy(
            out_v[bi], out_hbm.at[pl.ds(tok * M, RPC)], so[bi])

    def g_copies(bi):
        return [
            pltpu.make_async_copy(
                w_hbm.at[idx_v[bi].at[pl.ds(o, l)]],
                rows_v[bi].at[pl.ds(o, l)], sg[bi])
            for (o, l) in SPLITS
        ]

    pltpu.sync_copy(b_hbm, bias_v)

    pltpu.sync_copy(idx_hbm.at[pl.ds(tok_base * M, RPC)], idx_v[0])
    for cp in g_copies(0):
        cp.start()
    idx_copy(1, tok_base + T).start()
    x_copy(0, tok_base).start()
    x_copy(1, tok_base + T).start()

    lane16 = lane * 16

    def compute_chunk(bi):
        rows = rows_v[bi]
        xv = x_v[bi]

        @plsc.parallel_loop(0, T, 1, unroll=2)
        def tok_body(t):
            xa = xv[t, pl.ds(0, 32)]
            xb = xv[t, pl.ds(32, 32)]
            accs = []
            for m in range(M):
                r = t * M + m
                p = (xa * rows[r, pl.ds(0, 32)]
                     + xb * rows[r, pl.ds(32, 32)])
                lo, hi = plsc.unpack(p, format=plsc.PackFormat.INTERLEAVED)
                accs.append(lo + hi)
            for m in range(M):
                acc_t[pl.ds((t * M + m) * 16, 16)] = accs[m]

        @plsc.parallel_loop(0, NGRP, 1, unroll=2)
        def grp_body(g):
            vbase = lane16 + g * 256
            vals = [plsc.load_gather(acc_t, [vbase + l]) for l in range(16)]
            while len(vals) > 1:
                vals = [vals[i] + vals[i + 1] for i in range(0, len(vals), 2)]
            r0 = g * 16
            biasvals = plsc.load_gather(bias_v, [idx_v[bi][pl.ds(r0, 16)]])
            out_v[bi][pl.ds(r0, 16)] = jnp.maximum(vals[0] + biasvals, 0.0)

    def pair_body(pp, carry):
        for b in (0, 1):
            nb = 1 - b
            c = 2 * pp + b
            tok_c = tok_base + c * T

            def fire_next():
                idx_copy(nb, tok_c + T).wait()
                for cp in g_copies(nb):
                    cp.start()
            if b == 0:
                fire_next()
            else:
                pl.when(pp < CHUNKS // 2 - 1)(fire_next)

            for cp in g_copies(b):
                cp.wait()

            pl.when(pp > 0)(lambda: out_copy(b, tok_c - 2 * T).wait())

            x_copy(b, tok_c).wait()

            compute_chunk(b)
            out_copy(b, tok_c).start()

            def prefetch():
                idx_copy(b, tok_c + 2 * T).start()
                x_copy(b, tok_c + 2 * T).start()
            pl.when(pp < CHUNKS // 2 - 1)(prefetch)
        return carry

    lax.fori_loop(0, CHUNKS // 2, pair_body, 0)

    out_copy(0, tok_base + (CHUNKS - 2) * T).wait()
    out_copy(1, tok_base + (CHUNKS - 1) * T).wait()


@jax.jit
def _run(x2d, idx1d, weight, bias):
    f = pl.kernel(
        _sc_kernel,
        out_type=jax.ShapeDtypeStruct((N * M,), jnp.float32),
        mesh=plsc.VectorSubcoreMesh(core_axis_name="c", subcore_axis_name="s"),
        compiler_params=pltpu.CompilerParams(
            needs_layout_passes=False, use_tc_tiling_on_sc=False),
        scratch_types=[
            pltpu.VMEM((V,), jnp.float32),
            pltpu.VMEM((RPC,), jnp.int32),
            pltpu.VMEM((RPC,), jnp.int32),
            pltpu.VMEM((RPC, H), jnp.bfloat16),
            pltpu.VMEM((RPC, H), jnp.bfloat16),
            pltpu.VMEM((T, H), jnp.bfloat16),
            pltpu.VMEM((T, H), jnp.bfloat16),
            pltpu.VMEM((RPC,), jnp.float32),
            pltpu.VMEM((RPC,), jnp.float32),
            pltpu.VMEM((16 * RPC,), jnp.float32),
        ] + [pltpu.SemaphoreType.DMA] * 8,
    )
    return f(x2d, idx1d, weight, bias)


def kernel(input_value, mask_tensor, weight, bias):
    x2d = input_value.reshape(N, H).astype(jnp.bfloat16)
    idx1d = jnp.asarray(mask_tensor, jnp.int32).reshape(N * M)
    w_bf = weight.astype(jnp.bfloat16)
    out = _run(x2d, idx1d, w_bf, bias)
    return out.reshape(B, S, M)


# speedup vs baseline: 40.9114x; 1.0884x over previous
import jax
import jax.numpy as jnp
from jax import lax
from jax.experimental import pallas as pl
from jax.experimental.pallas import tpu as pltpu
from jax.experimental.pallas import tpu_sc as plsc

B, S, M, H, V = 1024, 50, 20, 64, 100000
N = B * S
NC, NS = 2, 16
NW = NC * NS
TOK_W = N // NW
T = 16
CHUNKS = TOK_W // T
RPC = T * M
SPLITS = ((0, 128), (128, 128), (256, 64))
NGRP = RPC // 16


def _sc_kernel(x_hbm, idx_hbm, w_hbm, b_hbm, out_hbm,
               bias_v, idx0, idx1, rows0, rows1, x0, x1, out0, out1, acc_t,
               sg0, sg1, si0, si1, sx0, sx1, so0, so1):
    c_id = lax.axis_index("c")
    s_id = lax.axis_index("s")
    wid = s_id * NC + c_id
    tok_base = wid * TOK_W
    lane = lax.iota(jnp.int32, 16)

    idx_v = (idx0, idx1)
    rows_v = (rows0, rows1)
    x_v = (x0, x1)
    out_v = (out0, out1)
    sg = (sg0, sg1)
    si = (si0, si1)
    sx = (sx0, sx1)
    so = (so0, so1)

    def idx_copy(bi, tok):
        return pltpu.make_async_copy(
            idx_hbm.at[pl.ds(tok * M, RPC)], idx_v[bi], si[bi])

    def x_copy(bi, tok):
        return pltpu.make_async_copy(
            x_hbm.at[pl.ds(tok, T)], x_v[bi], sx[bi])

    def out_copy(bi, tok):
        return pltpu.make_async_copy(
            out_v[bi], out_hbm.at[pl.ds(tok * M, RPC)], so[bi])

    def g_copies(bi):
        return [
            pltpu.make_async_copy(
                w_hbm.at[idx_v[bi].at[pl.ds(o, l)]],
                rows_v[bi].at[pl.ds(o, l)], sg[bi])
            for (o, l) in SPLITS
        ]

    pltpu.sync_copy(b_hbm, bias_v)

    pltpu.sync_copy(idx_hbm.at[pl.ds(tok_base * M, RPC)], idx_v[0])
    for cp in g_copies(0):
        cp.start()
    idx_copy(1, tok_base + T).start()
    x_copy(0, tok_base).start()
    x_copy(1, tok_base + T).start()

    lane16 = lane * 16

    def compute_chunk(bi):
        rows = rows_v[bi]
        xv = x_v[bi]

        @plsc.parallel_loop(0, T, 1, unroll=4)
        def tok_body(t):
            xa = xv[t, pl.ds(0, 32)]
            xb = xv[t, pl.ds(32, 32)]
            for m in range(M):
                r = t * M + m
                p = (xa * rows[r, pl.ds(0, 32)]
                     + xb * rows[r, pl.ds(32, 32)])
                lo, hi = plsc.unpack(p, format=plsc.PackFormat.INTERLEAVED)
                acc_t[pl.ds(r * 16, 16)] = lo + hi

        @plsc.parallel_loop(0, NGRP, 1, unroll=2)
        def grp_body(g):
            vbase = lane16 + g * 256
            vals = [plsc.load_gather(acc_t, [vbase + l]) for l in range(16)]
            while len(vals) > 1:
                vals = [vals[i] + vals[i + 1] for i in range(0, len(vals), 2)]
            r0 = g * 16
            biasvals = plsc.load_gather(bias_v, [idx_v[bi][pl.ds(r0, 16)]])
            out_v[bi][pl.ds(r0, 16)] = jnp.maximum(vals[0] + biasvals, 0.0)

    def pair_body(pp, carry):
        for b in (0, 1):
            nb = 1 - b
            c = 2 * pp + b
            tok_c = tok_base + c * T

            def fire_next():
                idx_copy(nb, tok_c + T).wait()
                for cp in g_copies(nb):
                    cp.start()
            if b == 0:
                fire_next()
            else:
                pl.when(pp < CHUNKS // 2 - 1)(fire_next)

            for cp in g_copies(b):
                cp.wait()

            pl.when(pp > 0)(lambda: out_copy(b, tok_c - 2 * T).wait())

            x_copy(b, tok_c).wait()

            compute_chunk(b)
            out_copy(b, tok_c).start()

            def prefetch():
                idx_copy(b, tok_c + 2 * T).start()
                x_copy(b, tok_c + 2 * T).start()
            pl.when(pp < CHUNKS // 2 - 1)(prefetch)
        return carry

    lax.fori_loop(0, CHUNKS // 2, pair_body, 0)

    out_copy(0, tok_base + (CHUNKS - 2) * T).wait()
    out_copy(1, tok_base + (CHUNKS - 1) * T).wait()


@jax.jit
def _run(x2d, idx1d, weight, bias):
    f = pl.kernel(
        _sc_kernel,
        out_type=jax.ShapeDtypeStruct((N * M,), jnp.float32),
        mesh=plsc.VectorSubcoreMesh(core_axis_name="c", subcore_axis_name="s"),
        compiler_params=pltpu.CompilerParams(
            needs_layout_passes=False, use_tc_tiling_on_sc=False),
        scratch_types=[
            pltpu.VMEM((V,), jnp.float32),
            pltpu.VMEM((RPC,), jnp.int32),
            pltpu.VMEM((RPC,), jnp.int32),
            pltpu.VMEM((RPC, H), jnp.bfloat16),
            pltpu.VMEM((RPC, H), jnp.bfloat16),
            pltpu.VMEM((T, H), jnp.bfloat16),
            pltpu.VMEM((T, H), jnp.bfloat16),
            pltpu.VMEM((RPC,), jnp.float32),
            pltpu.VMEM((RPC,), jnp.float32),
            pltpu.VMEM((16 * RPC,), jnp.float32),
        ] + [pltpu.SemaphoreType.DMA] * 8,
    )
    return f(x2d, idx1d, weight, bias)


def kernel(input_value, mask_tensor, weight, bias):
    x2d = input_value.reshape(N, H).astype(jnp.bfloat16)
    idx1d = jnp.asarray(mask_tensor, jnp.int32).reshape(N * M)
    w_bf = weight.astype(jnp.bfloat16)
    out = _run(x2d, idx1d, w_bf, bias)
    return out.reshape(B, S, M)
